# baseline (speedup 1.0000x reference)
import jax
import jax.numpy as jnp
from jax import lax
from jax.experimental import pallas as pl
from jax.experimental.pallas import tpu as pltpu
from jax.experimental.pallas import tpu_sc as plsc

_N = 100000
_D = 128
_NC = 2
_NS = 16
_NW = _NC * _NS

_N_SC = 54000
_N_TC = _N - _N_SC
_RPW = 1688
_CHUNK = 192
_NCHUNKS = 9
_UNROLL = 8

_BT = 2000
_GRID_TC = _N_TC // _BT
_BIG_I32 = 2147483647


def _sc_body(u_hbm, loc_hbm, dist_hbm, idx_hbm, u_v, buf, resd, resi,
             sem0, sem1, sem2, sem3):
    wid = lax.axis_index("s") * _NC + lax.axis_index("c")
    base = wid * _RPW

    pltpu.sync_copy(u_hbm, u_v)
    u_regs = [u_v[pl.ds(16 * j, 16)] for j in range(8)]
    sems = (sem0, sem1, sem2, sem3)
    _SLOT = _CHUNK * _D

    lane = lax.iota(jnp.int32, 16)
    rots = [(lane + k) & 15 for k in (1, 2, 4, 8)]

    def hsum(v):
        for idx in rots:
            v = v + v[idx]
        return v

    def start_slot(c, b):
        off = pl.multiple_of((base + c * _CHUNK) * _D, _D)
        pltpu.make_async_copy(
            loc_hbm.at[pl.ds(off, _SLOT)],
            buf.at[pl.ds(b * _SLOT, _SLOT)], sems[b]
        ).start()

    def start(c):
        for b in range(4):
            @pl.when((c & 3) == b)
            def _():
                start_slot(c, b)

    def wait(c):
        for b in range(4):
            @pl.when((c & 3) == b)
            def _():
                pltpu.make_async_copy(
                    loc_hbm.at[pl.ds(0, _SLOT)],
                    buf.at[pl.ds(b * _SLOT, _SLOT)], sems[b]
                ).wait()

    def process(off, row0, carry):
        def body(k, carry):
            pairs = list(carry)
            for t in range(_UNROLL):
                r = k * _UNROLL + t
                sq = []
                for j in range(8):
                    dif = buf[pl.ds(off + r * _D + 16 * j, 16)] - u_regs[j]
                    sq.append(dif * dif)
                while len(sq) > 1:
                    sq = [a + b2 for a, b2 in zip(sq[::2], sq[1::2])]
                d = hsum(sq[0])
                bd, bi = pairs[t]
                better = d < bd
                bd = jnp.where(better, d, bd)
                bi = jnp.where(better, jnp.full((16,), row0 + r, jnp.int32),
                               bi)
                pairs[t] = (bd, bi)
            return tuple(pairs)

        return plsc.parallel_loop(0, _CHUNK // _UNROLL, carry=carry)(body)

    for c in range(3):
        start_slot(c, c)
    init = tuple((jnp.full((16,), jnp.inf, jnp.float32),
                  jnp.zeros((16,), jnp.int32)) for _ in range(_UNROLL))

    def outer(c, carry):
        @pl.when(c + 3 < _NCHUNKS)
        def _():
            start(c + 3)

        wait(c)
        off = (c & 3) * _SLOT
        return process(off, base + c * _CHUNK, carry)

    carry = lax.fori_loop(0, _NCHUNKS, outer, init)

    pairs = list(carry)
    while len(pairs) > 1:
        nxt = []
        for (da, ia), (db, ib) in zip(pairs[::2], pairs[1::2]):
            take_b = (db < da) | ((db == da) & (ib < ia))
            nxt.append((jnp.where(take_b, db, da),
                        jnp.where(take_b, ib, ia)))
        pairs = nxt
    best_d, best_i = pairs[0]

    resd[...] = best_d
    resi[...] = best_i
    pltpu.sync_copy(resd, dist_hbm.at[pl.ds(wid * 16, 16)])
    pltpu.sync_copy(resi, idx_hbm.at[pl.ds(wid * 16, 16)])


def _tc_body(u_ref, loc_ref, tcd_ref, tci_ref, runmin, runidx):
    pid = pl.program_id(0)

    @pl.when(pid == 0)
    def _():
        runmin[...] = jnp.full((1, _BT), jnp.inf, jnp.float32)
        runidx[...] = jnp.zeros((1, _BT), jnp.int32)

    diff = loc_ref[...] - u_ref[...]
    dn = (((1,), (1,)), ((), ()))
    d2 = lax.dot_general(jnp.ones((1, _D), jnp.float32), diff * diff,
                         dn, preferred_element_type=jnp.float32)
    idxs = (_N_SC + pid * _BT
            + lax.broadcasted_iota(jnp.int32, (1, _BT), 1))
    better = d2 < runmin[...]
    runmin[...] = jnp.where(better, d2, runmin[...])
    runidx[...] = jnp.where(better, idxs, runidx[...])

    @pl.when(pid == _GRID_TC - 1)
    def _():
        mn = jnp.min(runmin[...])
        tcd_ref[0] = mn
        tci_ref[0] = jnp.min(jnp.where(runmin[...] == mn, runidx[...],
                                       _BIG_I32))


def _merge_body(scd_ref, sci_ref, tcd_ref, tci_ref, loc_hbm, out_ref,
                row_v, sem):
    scd = scd_ref[...]
    scmn = jnp.min(scd)
    scix = jnp.min(jnp.where(scd == scmn, sci_ref[...], _BIG_I32))
    use_sc = scmn <= tcd_ref[0]
    bi = jnp.where(use_sc, scix, tci_ref[0])
    off = pl.multiple_of(bi * _D, _D)
    copy = pltpu.make_async_copy(loc_hbm.at[pl.ds(off, _D)], row_v, sem)
    copy.start()
    copy.wait()
    out_ref[...] = row_v[...]


def kernel(u, loc, p):
    del p
    loc_flat = loc.reshape(-1)

    sc_stage = pl.kernel(
        _sc_body,
        out_type=[
            jax.ShapeDtypeStruct((_NW * 16,), jnp.float32),
            jax.ShapeDtypeStruct((_NW * 16,), jnp.int32),
        ],
        mesh=plsc.VectorSubcoreMesh(core_axis_name="c", subcore_axis_name="s"),
        scratch_types=[
            pltpu.VMEM((_D,), jnp.float32),
            pltpu.VMEM((4 * _CHUNK * _D,), jnp.float32),
            pltpu.VMEM((16,), jnp.float32),
            pltpu.VMEM((16,), jnp.int32),
            pltpu.SemaphoreType.DMA,
            pltpu.SemaphoreType.DMA,
            pltpu.SemaphoreType.DMA,
            pltpu.SemaphoreType.DMA,
        ],
    )
    sc_d, sc_i = sc_stage(u, loc_flat)

    tc_d, tc_i = pl.pallas_call(
        _tc_body,
        grid=(_GRID_TC,),
        in_specs=[
            pl.BlockSpec((1, _D), lambda i: (0, 0)),
            pl.BlockSpec((_BT, _D), lambda i: (i + _N_SC // _BT, 0)),
        ],
        out_specs=[
            pl.BlockSpec(memory_space=pltpu.SMEM),
            pl.BlockSpec(memory_space=pltpu.SMEM),
        ],
        out_shape=[
            jax.ShapeDtypeStruct((1,), jnp.float32),
            jax.ShapeDtypeStruct((1,), jnp.int32),
        ],
        scratch_shapes=[
            pltpu.VMEM((1, _BT), jnp.float32),
            pltpu.VMEM((1, _BT), jnp.int32),
        ],
    )(u.reshape(1, _D), loc)

    out = pl.pallas_call(
        _merge_body,
        in_specs=[
            pl.BlockSpec(memory_space=pltpu.VMEM),
            pl.BlockSpec(memory_space=pltpu.VMEM),
            pl.BlockSpec(memory_space=pltpu.SMEM),
            pl.BlockSpec(memory_space=pltpu.SMEM),
            pl.BlockSpec(memory_space=pl.ANY),
        ],
        out_specs=pl.BlockSpec(memory_space=pltpu.VMEM),
        out_shape=jax.ShapeDtypeStruct((_D,), jnp.float32),
        scratch_shapes=[
            pltpu.VMEM((_D,), jnp.float32),
            pltpu.SemaphoreType.DMA,
        ],
    )(sc_d, sc_i, tc_d, tc_i, loc_flat)
    return out


# speedup vs baseline: 1.0347x; 1.0347x over previous
import jax
import jax.numpy as jnp
from jax import lax
from jax.experimental import pallas as pl
from jax.experimental.pallas import tpu as pltpu
from jax.experimental.pallas import tpu_sc as plsc

_N = 100000
_D = 128
_NC = 2
_NS = 16
_NW = _NC * _NS

_N_SC = 52000
_N_TC = _N - _N_SC
_RPW = 1625
_CHUNK = 128
_NCHUNKS = 13
_UNROLL = 8

_BT = 4000
_GRID_TC = _N_TC // _BT
_BIG_I32 = 2147483647


def _sc_body(u_hbm, loc_hbm, dist_hbm, idx_hbm, u_v, buf, resd, resi,
             sem0, sem1, sem2, sem3):
    wid = lax.axis_index("s") * _NC + lax.axis_index("c")
    base = wid * _RPW

    pltpu.sync_copy(u_hbm, u_v)
    u_regs = [u_v[pl.ds(16 * j, 16)] for j in range(8)]
    sems = (sem0, sem1, sem2, sem3)
    _SLOT = _CHUNK * _D

    lane = lax.iota(jnp.int32, 16)
    rots = [(lane + k) & 15 for k in (1, 2, 4, 8)]

    def hsum(v):
        for idx in rots:
            v = v + v[idx]
        return v

    def start_slot(c, b):
        off = pl.multiple_of((base + c * _CHUNK) * _D, _D)
        pltpu.make_async_copy(
            loc_hbm.at[pl.ds(off, _SLOT)],
            buf.at[pl.ds(b * _SLOT, _SLOT)], sems[b]
        ).start()

    def start(c):
        for b in range(4):
            @pl.when((c & 3) == b)
            def _():
                start_slot(c, b)

    def wait(c):
        for b in range(4):
            @pl.when((c & 3) == b)
            def _():
                pltpu.make_async_copy(
                    loc_hbm.at[pl.ds(0, _SLOT)],
                    buf.at[pl.ds(b * _SLOT, _SLOT)], sems[b]
                ).wait()

    def process(off, row0, carry):
        def body(k, carry):
            pairs = list(carry)
            for t in range(_UNROLL):
                r = k * _UNROLL + t
                sq = []
                for j in range(8):
                    dif = buf[pl.ds(off + r * _D + 16 * j, 16)] - u_regs[j]
                    sq.append(dif * dif)
                while len(sq) > 1:
                    sq = [a + b2 for a, b2 in zip(sq[::2], sq[1::2])]
                d = hsum(sq[0])
                bd, bi = pairs[t]
                better = d < bd
                bd = jnp.where(better, d, bd)
                bi = jnp.where(better, jnp.full((16,), row0 + r, jnp.int32),
                               bi)
                pairs[t] = (bd, bi)
            return tuple(pairs)

        return plsc.parallel_loop(0, _CHUNK // _UNROLL, carry=carry)(body)

    for c in range(3):
        start_slot(c, c)
    init = tuple((jnp.full((16,), jnp.inf, jnp.float32),
                  jnp.zeros((16,), jnp.int32)) for _ in range(_UNROLL))

    def outer(c, carry):
        @pl.when(c + 3 < _NCHUNKS)
        def _():
            start(c + 3)

        wait(c)
        off = (c & 3) * _SLOT
        return process(off, base + c * _CHUNK, carry)

    carry = lax.fori_loop(0, _NCHUNKS, outer, init)

    pairs = list(carry)
    while len(pairs) > 1:
        nxt = []
        for (da, ia), (db, ib) in zip(pairs[::2], pairs[1::2]):
            take_b = (db < da) | ((db == da) & (ib < ia))
            nxt.append((jnp.where(take_b, db, da),
                        jnp.where(take_b, ib, ia)))
        pairs = nxt
    best_d, best_i = pairs[0]

    resd[...] = best_d
    resi[...] = best_i
    pltpu.sync_copy(resd, dist_hbm.at[pl.ds(wid * 16, 16)])
    pltpu.sync_copy(resi, idx_hbm.at[pl.ds(wid * 16, 16)])


def _tc_body(u_ref, loc_ref, tcd_ref, tci_ref, runmin, runidx):
    pid = pl.program_id(0)

    @pl.when(pid == 0)
    def _():
        runmin[...] = jnp.full((1, _BT), jnp.inf, jnp.float32)
        runidx[...] = jnp.zeros((1, _BT), jnp.int32)

    diff = loc_ref[...] - u_ref[...]
    dn = (((1,), (1,)), ((), ()))
    d2 = lax.dot_general(jnp.ones((1, _D), jnp.float32), diff * diff,
                         dn, preferred_element_type=jnp.float32)
    idxs = (_N_SC + pid * _BT
            + lax.broadcasted_iota(jnp.int32, (1, _BT), 1))
    better = d2 < runmin[...]
    runmin[...] = jnp.where(better, d2, runmin[...])
    runidx[...] = jnp.where(better, idxs, runidx[...])

    @pl.when(pid == _GRID_TC - 1)
    def _():
        mn = jnp.min(runmin[...])
        tcd_ref[0] = mn
        tci_ref[0] = jnp.min(jnp.where(runmin[...] == mn, runidx[...],
                                       _BIG_I32))


def _merge_body(scd_ref, sci_ref, tcd_ref, tci_ref, loc_hbm, out_ref,
                row_v, sem):
    scd = scd_ref[...]
    scmn = jnp.min(scd)
    scix = jnp.min(jnp.where(scd == scmn, sci_ref[...], _BIG_I32))
    use_sc = scmn <= tcd_ref[0]
    bi = jnp.where(use_sc, scix, tci_ref[0])
    off = pl.multiple_of(bi * _D, _D)
    copy = pltpu.make_async_copy(loc_hbm.at[pl.ds(off, _D)], row_v, sem)
    copy.start()
    copy.wait()
    out_ref[...] = row_v[...]


def kernel(u, loc, p):
    del p
    loc_flat = loc.reshape(-1)

    sc_stage = pl.kernel(
        _sc_body,
        out_type=[
            jax.ShapeDtypeStruct((_NW * 16,), jnp.float32),
            jax.ShapeDtypeStruct((_NW * 16,), jnp.int32),
        ],
        mesh=plsc.VectorSubcoreMesh(core_axis_name="c", subcore_axis_name="s"),
        scratch_types=[
            pltpu.VMEM((_D,), jnp.float32),
            pltpu.VMEM((4 * _CHUNK * _D,), jnp.float32),
            pltpu.VMEM((16,), jnp.float32),
            pltpu.VMEM((16,), jnp.int32),
            pltpu.SemaphoreType.DMA,
            pltpu.SemaphoreType.DMA,
            pltpu.SemaphoreType.DMA,
            pltpu.SemaphoreType.DMA,
        ],
    )
    sc_d, sc_i = sc_stage(u, loc_flat)

    tc_d, tc_i = pl.pallas_call(
        _tc_body,
        grid=(_GRID_TC,),
        in_specs=[
            pl.BlockSpec((1, _D), lambda i: (0, 0)),
            pl.BlockSpec((_BT, _D), lambda i: (i + _N_SC // _BT, 0)),
        ],
        out_specs=[
            pl.BlockSpec(memory_space=pltpu.SMEM),
            pl.BlockSpec(memory_space=pltpu.SMEM),
        ],
        out_shape=[
            jax.ShapeDtypeStruct((1,), jnp.float32),
            jax.ShapeDtypeStruct((1,), jnp.int32),
        ],
        scratch_shapes=[
            pltpu.VMEM((1, _BT), jnp.float32),
            pltpu.VMEM((1, _BT), jnp.int32),
        ],
    )(u.reshape(1, _D), loc)

    out = pl.pallas_call(
        _merge_body,
        in_specs=[
            pl.BlockSpec(memory_space=pltpu.VMEM),
            pl.BlockSpec(memory_space=pltpu.VMEM),
            pl.BlockSpec(memory_space=pltpu.SMEM),
            pl.BlockSpec(memory_space=pltpu.SMEM),
            pl.BlockSpec(memory_space=pl.ANY),
        ],
        out_specs=pl.BlockSpec(memory_space=pltpu.VMEM),
        out_shape=jax.ShapeDtypeStruct((_D,), jnp.float32),
        scratch_shapes=[
            pltpu.VMEM((_D,), jnp.float32),
            pltpu.SemaphoreType.DMA,
        ],
    )(sc_d, sc_i, tc_d, tc_i, loc_flat)
    return out


# speedup vs baseline: 1.0515x; 1.0163x over previous
import jax
import jax.numpy as jnp
from jax import lax
from jax.experimental import pallas as pl
from jax.experimental.pallas import tpu as pltpu
from jax.experimental.pallas import tpu_sc as plsc

_N = 100000
_D = 128
_NC = 2
_NS = 16
_NW = _NC * _NS

_N_SC = 52000
_N_TC = _N - _N_SC
_RPW = 1625
_CHUNK = 128
_NCHUNKS = 13
_UNROLL = 4

_BT = 4000
_GRID_TC = _N_TC // _BT
_BIG_I32 = 2147483647


def _sc_body(u_hbm, loc_hbm, dist_hbm, idx_hbm, u_v, buf, resd, resi,
             sem0, sem1, sem2, sem3):
    wid = lax.axis_index("s") * _NC + lax.axis_index("c")
    base = wid * _RPW

    pltpu.sync_copy(u_hbm, u_v)
    u_regs = [u_v[pl.ds(16 * j, 16)] for j in range(8)]
    sems = (sem0, sem1, sem2, sem3)
    _SLOT = _CHUNK * _D

    lane = lax.iota(jnp.int32, 16)
    rots = [(lane + k) & 15 for k in (1, 2, 4, 8)]

    def hsum(v):
        for idx in rots:
            v = v + v[idx]
        return v

    def start_slot(c, b):
        off = pl.multiple_of((base + c * _CHUNK) * _D, _D)
        pltpu.make_async_copy(
            loc_hbm.at[pl.ds(off, _SLOT)],
            buf.at[pl.ds(b * _SLOT, _SLOT)], sems[b]
        ).start()

    def start(c):
        for b in range(4):
            @pl.when((c & 3) == b)
            def _():
                start_slot(c, b)

    def wait(c):
        for b in range(4):
            @pl.when((c & 3) == b)
            def _():
                pltpu.make_async_copy(
                    loc_hbm.at[pl.ds(0, _SLOT)],
                    buf.at[pl.ds(b * _SLOT, _SLOT)], sems[b]
                ).wait()

    def process(off, row0, carry):
        def body(k, carry):
            pairs = list(carry)
            for t in range(_UNROLL):
                r = k * _UNROLL + t
                sq = []
                for j in range(8):
                    dif = buf[pl.ds(off + r * _D + 16 * j, 16)] - u_regs[j]
                    sq.append(dif * dif)
                while len(sq) > 1:
                    sq = [a + b2 for a, b2 in zip(sq[::2], sq[1::2])]
                d = hsum(sq[0])
                bd, bi = pairs[t]
                better = d < bd
                bd = jnp.where(better, d, bd)
                bi = jnp.where(better, jnp.full((16,), row0 + r, jnp.int32),
                               bi)
                pairs[t] = (bd, bi)
            return tuple(pairs)

        return plsc.parallel_loop(0, _CHUNK // _UNROLL, carry=carry)(body)

    for c in range(3):
        start_slot(c, c)
    init = tuple((jnp.full((16,), jnp.inf, jnp.float32),
                  jnp.zeros((16,), jnp.int32)) for _ in range(_UNROLL))

    def outer(c, carry):
        @pl.when(c + 3 < _NCHUNKS)
        def _():
            start(c + 3)

        wait(c)
        off = (c & 3) * _SLOT
        return process(off, base + c * _CHUNK, carry)

    carry = lax.fori_loop(0, _NCHUNKS, outer, init)

    pairs = list(carry)
    while len(pairs) > 1:
        nxt = []
        for (da, ia), (db, ib) in zip(pairs[::2], pairs[1::2]):
            take_b = (db < da) | ((db == da) & (ib < ia))
            nxt.append((jnp.where(take_b, db, da),
                        jnp.where(take_b, ib, ia)))
        pairs = nxt
    best_d, best_i = pairs[0]

    resd[...] = best_d
    resi[...] = best_i
    pltpu.sync_copy(resd, dist_hbm.at[pl.ds(wid * 16, 16)])
    pltpu.sync_copy(resi, idx_hbm.at[pl.ds(wid * 16, 16)])


def _tc_body(u_ref, loc_ref, tcd_ref, tci_ref, runmin, runidx):
    pid = pl.program_id(0)

    @pl.when(pid == 0)
    def _():
        runmin[...] = jnp.full((1, _BT), jnp.inf, jnp.float32)
        runidx[...] = jnp.zeros((1, _BT), jnp.int32)

    diff = loc_ref[...] - u_ref[...]
    dn = (((1,), (1,)), ((), ()))
    d2 = lax.dot_general(jnp.ones((1, _D), jnp.float32), diff * diff,
                         dn, preferred_element_type=jnp.float32)
    idxs = (_N_SC + pid * _BT
            + lax.broadcasted_iota(jnp.int32, (1, _BT), 1))
    better = d2 < runmin[...]
    runmin[...] = jnp.where(better, d2, runmin[...])
    runidx[...] = jnp.where(better, idxs, runidx[...])

    @pl.when(pid == _GRID_TC - 1)
    def _():
        mn = jnp.min(runmin[...])
        tcd_ref[0] = mn
        tci_ref[0] = jnp.min(jnp.where(runmin[...] == mn, runidx[...],
                                       _BIG_I32))


def _merge_body(scd_ref, sci_ref, tcd_ref, tci_ref, loc_hbm, out_ref,
                row_v, sem):
    scd = scd_ref[...]
    scmn = jnp.min(scd)
    scix = jnp.min(jnp.where(scd == scmn, sci_ref[...], _BIG_I32))
    use_sc = scmn <= tcd_ref[0]
    bi = jnp.where(use_sc, scix, tci_ref[0])
    off = pl.multiple_of(bi * _D, _D)
    copy = pltpu.make_async_copy(loc_hbm.at[pl.ds(off, _D)], row_v, sem)
    copy.start()
    copy.wait()
    out_ref[...] = row_v[...]


def kernel(u, loc, p):
    del p
    loc_flat = loc.reshape(-1)

    sc_stage = pl.kernel(
        _sc_body,
        out_type=[
            jax.ShapeDtypeStruct((_NW * 16,), jnp.float32),
            jax.ShapeDtypeStruct((_NW * 16,), jnp.int32),
        ],
        mesh=plsc.VectorSubcoreMesh(core_axis_name="c", subcore_axis_name="s"),
        scratch_types=[
            pltpu.VMEM((_D,), jnp.float32),
            pltpu.VMEM((4 * _CHUNK * _D,), jnp.float32),
            pltpu.VMEM((16,), jnp.float32),
            pltpu.VMEM((16,), jnp.int32),
            pltpu.SemaphoreType.DMA,
            pltpu.SemaphoreType.DMA,
            pltpu.SemaphoreType.DMA,
            pltpu.SemaphoreType.DMA,
        ],
    )
    sc_d, sc_i = sc_stage(u, loc_flat)

    tc_d, tc_i = pl.pallas_call(
        _tc_body,
        grid=(_GRID_TC,),
        in_specs=[
            pl.BlockSpec((1, _D), lambda i: (0, 0)),
            pl.BlockSpec((_BT, _D), lambda i: (i + _N_SC // _BT, 0)),
        ],
        out_specs=[
            pl.BlockSpec(memory_space=pltpu.SMEM),
            pl.BlockSpec(memory_space=pltpu.SMEM),
        ],
        out_shape=[
            jax.ShapeDtypeStruct((1,), jnp.float32),
            jax.ShapeDtypeStruct((1,), jnp.int32),
        ],
        scratch_shapes=[
            pltpu.VMEM((1, _BT), jnp.float32),
            pltpu.VMEM((1, _BT), jnp.int32),
        ],
    )(u.reshape(1, _D), loc)

    out = pl.pallas_call(
        _merge_body,
        in_specs=[
            pl.BlockSpec(memory_space=pltpu.VMEM),
            pl.BlockSpec(memory_space=pltpu.VMEM),
            pl.BlockSpec(memory_space=pltpu.SMEM),
            pl.BlockSpec(memory_space=pltpu.SMEM),
            pl.BlockSpec(memory_space=pl.ANY),
        ],
        out_specs=pl.BlockSpec(memory_space=pltpu.VMEM),
        out_shape=jax.ShapeDtypeStruct((_D,), jnp.float32),
        scratch_shapes=[
            pltpu.VMEM((_D,), jnp.float32),
            pltpu.SemaphoreType.DMA,
        ],
    )(sc_d, sc_i, tc_d, tc_i, loc_flat)
    return out


# speedup vs baseline: 1.0628x; 1.0108x over previous
import jax
import jax.numpy as jnp
from jax import lax
from jax.experimental import pallas as pl
from jax.experimental.pallas import tpu as pltpu
from jax.experimental.pallas import tpu_sc as plsc

_N = 100000
_D = 128
_NC = 2
_NS = 16
_NW = _NC * _NS

_N_SC = 48000
_N_TC = _N - _N_SC
_RPW = 1500
_CHUNK = 128
_NCHUNKS = 12
_UNROLL = 4

_BT = 4000
_GRID_TC = _N_TC // _BT
_BIG_I32 = 2147483647


def _sc_body(u_hbm, loc_hbm, dist_hbm, idx_hbm, u_v, buf, resd, resi,
             sem0, sem1, sem2, sem3):
    wid = lax.axis_index("s") * _NC + lax.axis_index("c")
    base = wid * _RPW

    pltpu.sync_copy(u_hbm, u_v)
    u_regs = [u_v[pl.ds(16 * j, 16)] for j in range(8)]
    sems = (sem0, sem1, sem2, sem3)
    _SLOT = _CHUNK * _D

    lane = lax.iota(jnp.int32, 16)
    rots = [(lane + k) & 15 for k in (1, 2, 4, 8)]

    def hsum(v):
        for idx in rots:
            v = v + v[idx]
        return v

    def start_slot(c, b):
        off = pl.multiple_of((base + c * _CHUNK) * _D, _D)
        pltpu.make_async_copy(
            loc_hbm.at[pl.ds(off, _SLOT)],
            buf.at[pl.ds(b * _SLOT, _SLOT)], sems[b]
        ).start()

    def start(c):
        for b in range(4):
            @pl.when((c & 3) == b)
            def _():
                start_slot(c, b)

    def wait(c):
        for b in range(4):
            @pl.when((c & 3) == b)
            def _():
                pltpu.make_async_copy(
                    loc_hbm.at[pl.ds(0, _SLOT)],
                    buf.at[pl.ds(b * _SLOT, _SLOT)], sems[b]
                ).wait()

    def process(off, row0, carry):
        def body(k, carry):
            pairs = list(carry)
            for t in range(_UNROLL):
                r = k * _UNROLL + t
                sq = []
                for j in range(8):
                    dif = buf[pl.ds(off + r * _D + 16 * j, 16)] - u_regs[j]
                    sq.append(dif * dif)
                while len(sq) > 1:
                    sq = [a + b2 for a, b2 in zip(sq[::2], sq[1::2])]
                d = hsum(sq[0])
                bd, bi = pairs[t]
                better = d < bd
                bd = jnp.where(better, d, bd)
                bi = jnp.where(better, jnp.full((16,), row0 + r, jnp.int32),
                               bi)
                pairs[t] = (bd, bi)
            return tuple(pairs)

        return plsc.parallel_loop(0, _CHUNK // _UNROLL, carry=carry)(body)

    for c in range(3):
        start_slot(c, c)
    init = tuple((jnp.full((16,), jnp.inf, jnp.float32),
                  jnp.zeros((16,), jnp.int32)) for _ in range(_UNROLL))

    def outer(c, carry):
        @pl.when(c + 3 < _NCHUNKS)
        def _():
            start(c + 3)

        wait(c)
        off = (c & 3) * _SLOT
        return process(off, base + c * _CHUNK, carry)

    carry = lax.fori_loop(0, _NCHUNKS, outer, init)

    pairs = list(carry)
    while len(pairs) > 1:
        nxt = []
        for (da, ia), (db, ib) in zip(pairs[::2], pairs[1::2]):
            take_b = (db < da) | ((db == da) & (ib < ia))
            nxt.append((jnp.where(take_b, db, da),
                        jnp.where(take_b, ib, ia)))
        pairs = nxt
    best_d, best_i = pairs[0]

    resd[...] = best_d
    resi[...] = best_i
    pltpu.sync_copy(resd, dist_hbm.at[pl.ds(wid * 16, 16)])
    pltpu.sync_copy(resi, idx_hbm.at[pl.ds(wid * 16, 16)])


def _tc_body(u_ref, loc_ref, tcd_ref, tci_ref, runmin, runidx):
    pid = pl.program_id(0)

    @pl.when(pid == 0)
    def _():
        runmin[...] = jnp.full((1, _BT), jnp.inf, jnp.float32)
        runidx[...] = jnp.zeros((1, _BT), jnp.int32)

    diff = loc_ref[...] - u_ref[...]
    dn = (((1,), (1,)), ((), ()))
    d2 = lax.dot_general(jnp.ones((1, _D), jnp.float32), diff * diff,
                         dn, preferred_element_type=jnp.float32)
    idxs = (_N_SC + pid * _BT
            + lax.broadcasted_iota(jnp.int32, (1, _BT), 1))
    better = d2 < runmin[...]
    runmin[...] = jnp.where(better, d2, runmin[...])
    runidx[...] = jnp.where(better, idxs, runidx[...])

    @pl.when(pid == _GRID_TC - 1)
    def _():
        mn = jnp.min(runmin[...])
        tcd_ref[0] = mn
        tci_ref[0] = jnp.min(jnp.where(runmin[...] == mn, runidx[...],
                                       _BIG_I32))


def _merge_body(scd_ref, sci_ref, tcd_ref, tci_ref, loc_hbm, out_ref,
                row_v, sem):
    scd = scd_ref[...]
    scmn = jnp.min(scd)
    scix = jnp.min(jnp.where(scd == scmn, sci_ref[...], _BIG_I32))
    use_sc = scmn <= tcd_ref[0]
    bi = jnp.where(use_sc, scix, tci_ref[0])
    off = pl.multiple_of(bi * _D, _D)
    copy = pltpu.make_async_copy(loc_hbm.at[pl.ds(off, _D)], row_v, sem)
    copy.start()
    copy.wait()
    out_ref[...] = row_v[...]


def kernel(u, loc, p):
    del p
    loc_flat = loc.reshape(-1)

    sc_stage = pl.kernel(
        _sc_body,
        out_type=[
            jax.ShapeDtypeStruct((_NW * 16,), jnp.float32),
            jax.ShapeDtypeStruct((_NW * 16,), jnp.int32),
        ],
        mesh=plsc.VectorSubcoreMesh(core_axis_name="c", subcore_axis_name="s"),
        scratch_types=[
            pltpu.VMEM((_D,), jnp.float32),
            pltpu.VMEM((4 * _CHUNK * _D,), jnp.float32),
            pltpu.VMEM((16,), jnp.float32),
            pltpu.VMEM((16,), jnp.int32),
            pltpu.SemaphoreType.DMA,
            pltpu.SemaphoreType.DMA,
            pltpu.SemaphoreType.DMA,
            pltpu.SemaphoreType.DMA,
        ],
    )
    sc_d, sc_i = sc_stage(u, loc_flat)

    tc_d, tc_i = pl.pallas_call(
        _tc_body,
        grid=(_GRID_TC,),
        in_specs=[
            pl.BlockSpec((1, _D), lambda i: (0, 0)),
            pl.BlockSpec((_BT, _D), lambda i: (i + _N_SC // _BT, 0)),
        ],
        out_specs=[
            pl.BlockSpec(memory_space=pltpu.SMEM),
            pl.BlockSpec(memory_space=pltpu.SMEM),
        ],
        out_shape=[
            jax.ShapeDtypeStruct((1,), jnp.float32),
            jax.ShapeDtypeStruct((1,), jnp.int32),
        ],
        scratch_shapes=[
            pltpu.VMEM((1, _BT), jnp.float32),
            pltpu.VMEM((1, _BT), jnp.int32),
        ],
    )(u.reshape(1, _D), loc)

    out = pl.pallas_call(
        _merge_body,
        in_specs=[
            pl.BlockSpec(memory_space=pltpu.VMEM),
            pl.BlockSpec(memory_space=pltpu.VMEM),
            pl.BlockSpec(memory_space=pltpu.SMEM),
            pl.BlockSpec(memory_space=pltpu.SMEM),
            pl.BlockSpec(memory_space=pl.ANY),
        ],
        out_specs=pl.BlockSpec(memory_space=pltpu.VMEM),
        out_shape=jax.ShapeDtypeStruct((_D,), jnp.float32),
        scratch_shapes=[
            pltpu.VMEM((_D,), jnp.float32),
            pltpu.SemaphoreType.DMA,
        ],
    )(sc_d, sc_i, tc_d, tc_i, loc_flat)
    return out
